# Initial kernel scaffold; baseline (speedup 1.0000x reference)
import functools

import jax
import jax.numpy as jnp
from jax import lax
from jax.experimental import pallas as pl
from jax.experimental.pallas import tpu as pltpu
from jax.experimental.pallas import tpu_sc as plsc

EPS = 1e-5
K = 128


@functools.lru_cache(maxsize=None)
def _seg_sum_rows(T, S, C, D):
    info = plsc.get_sparse_core_info()
    NC, NS = info.num_cores, info.num_subcores
    NW = NC * NS
    base = C // NW
    rem = C - base * NW
    assert S % NS == 0
    rows_per = S // NS
    mesh = plsc.VectorSubcoreMesh(core_axis_name="c", subcore_axis_name="s")

    @functools.partial(
        pl.kernel,
        out_type=jax.ShapeDtypeStruct((NC, S, D), jnp.float32),
        mesh=mesh,
        scratch_types=[
            pltpu.VMEM((base + 1, K), jnp.int32),
            pltpu.VMEM((base + 1, K), jnp.int32),
            pltpu.VMEM((K, D), jnp.float32),
            pltpu.VMEM((K, D), jnp.float32),
            pltpu.VMEM_SHARED((S, D), jnp.float32),
            pltpu.SemaphoreType.DMA,
        ],
    )
    def k(tab, gidx, sidx, zeros, out, gbuf, sbuf, rowbuf, zbuf, acc, sem):
        c = lax.axis_index("c")
        s = lax.axis_index("s")
        w = s * NC + c

        pltpu.sync_copy(zeros, zbuf)
        row0 = s * rows_per
        for off in range(0, rows_per, K):
            sz = min(K, rows_per - off)
            pltpu.sync_copy(zbuf.at[pl.ds(0, sz)], acc.at[pl.ds(row0 + off, sz)])
        plsc.subcore_barrier()

        start = w * base + jnp.minimum(w, rem)
        pltpu.sync_copy(gidx.at[pl.ds(start, base)], gbuf.at[pl.ds(0, base)])
        pltpu.sync_copy(sidx.at[pl.ds(start, base)], sbuf.at[pl.ds(0, base)])

        @pl.when(w < rem)
        def _():
            pltpu.sync_copy(gidx.at[pl.ds(start + base, 1)], gbuf.at[pl.ds(base, 1)])
            pltpu.sync_copy(sidx.at[pl.ds(start + base, 1)], sbuf.at[pl.ds(base, 1)])

        def chunk(j):
            pltpu.async_copy(tab.at[gbuf.at[j]], rowbuf, sem).wait()
            pltpu.sync_copy(rowbuf, acc.at[sbuf.at[j]], add=True)

        def body(j, carry):
            chunk(j)
            return carry

        lax.fori_loop(0, base, body, 0)

        @pl.when(w < rem)
        def _():
            chunk(base)

        plsc.subcore_barrier()

        for off in range(0, rows_per, K):
            sz = min(K, rows_per - off)
            pltpu.sync_copy(acc.at[pl.ds(row0 + off, sz)], rowbuf.at[pl.ds(0, sz)])
            pltpu.sync_copy(rowbuf.at[pl.ds(0, sz)], out.at[c, pl.ds(row0 + off, sz)])

    return k


def _safe_inv(d):
    return jnp.where(d > 0, 1.0 / jnp.where(d > 0, d, 1.0), 0.0)


def _matmul_t(x, W):
    n, f = x.shape
    blk = 500
    assert n % blk == 0

    def body(x_ref, w_ref, o_ref):
        o_ref[...] = lax.dot_general(
            x_ref[...], w_ref[...], (((1,), (1,)), ((), ())),
            preferred_element_type=jnp.float32,
            precision=lax.Precision.HIGHEST)

    return pl.pallas_call(
        body,
        grid=(n // blk,),
        in_specs=[pl.BlockSpec((blk, f), lambda i: (i, 0)),
                  pl.BlockSpec((f, f), lambda i: (0, 0))],
        out_specs=pl.BlockSpec((blk, f), lambda i: (i, 0)),
        out_shape=jax.ShapeDtypeStruct((n, f), jnp.float32),
    )(x, W)


def _combine_scale(parts, deg_parts):
    _, s, d = parts.shape
    dd = deg_parts.shape[2]
    blk = 500
    assert s % blk == 0

    def body(p_ref, dg_ref, o_ref):
        deg = dg_ref[0, :, 0] + dg_ref[1, :, 0]
        o_ref[...] = (p_ref[0] + p_ref[1]) * _safe_inv(deg)[:, None]

    return pl.pallas_call(
        body,
        grid=(s // blk,),
        in_specs=[pl.BlockSpec((2, blk, d), lambda i: (0, i, 0)),
                  pl.BlockSpec((2, blk, dd), lambda i: (0, i, 0))],
        out_specs=pl.BlockSpec((blk, d), lambda i: (i, 0)),
        out_shape=jax.ShapeDtypeStruct((s, d), jnp.float32),
    )(parts, deg_parts)


def _combine_bias_tanh_stats(parts, deg_parts, b2):
    _, n, d = parts.shape
    dd = deg_parts.shape[2]
    blk = 500
    assert n % blk == 0

    def body(p_ref, dg_ref, b_ref, t_ref, s_ref):
        i = pl.program_id(0)
        deg = dg_ref[0, :, 0] + dg_ref[1, :, 0]
        z = (p_ref[0] + p_ref[1]) * _safe_inv(deg)[:, None] + b_ref[...]
        t = jnp.tanh(z)
        t_ref[...] = t
        st = jnp.concatenate(
            [jnp.sum(t, 0, keepdims=True),
             jnp.sum(t * t, 0, keepdims=True),
             jnp.zeros((6, d), jnp.float32)], axis=0)

        @pl.when(i == 0)
        def _():
            s_ref[...] = jnp.zeros_like(s_ref)

        s_ref[...] = s_ref[...] + st

    return pl.pallas_call(
        body,
        grid=(n // blk,),
        in_specs=[pl.BlockSpec((2, blk, d), lambda i: (0, i, 0)),
                  pl.BlockSpec((2, blk, dd), lambda i: (0, i, 0)),
                  pl.BlockSpec((1, d), lambda i: (0, 0))],
        out_specs=[pl.BlockSpec((blk, d), lambda i: (i, 0)),
                   pl.BlockSpec((8, d), lambda i: (0, 0))],
        out_shape=[jax.ShapeDtypeStruct((n, d), jnp.float32),
                   jax.ShapeDtypeStruct((8, d), jnp.float32)],
    )(parts, deg_parts, b2)


def _batchnorm_apply(t, sums, g2, beta2):
    n, d = t.shape
    blk = 500
    assert n % blk == 0
    inv_n = 1.0 / n

    def body(t_ref, s_ref, g_ref, be_ref, o_ref):
        m = s_ref[0, :] * inv_n
        v = s_ref[1, :] * inv_n - m * m
        scale = lax.rsqrt(v + EPS) * g_ref[0, :]
        o_ref[...] = (t_ref[...] - m[None, :]) * scale[None, :] + be_ref[...]

    return pl.pallas_call(
        body,
        grid=(n // blk,),
        in_specs=[pl.BlockSpec((blk, d), lambda i: (i, 0)),
                  pl.BlockSpec((8, d), lambda i: (0, 0)),
                  pl.BlockSpec((1, d), lambda i: (0, 0)),
                  pl.BlockSpec((1, d), lambda i: (0, 0))],
        out_specs=pl.BlockSpec((blk, d), lambda i: (i, 0)),
        out_shape=jax.ShapeDtypeStruct((n, d), jnp.float32),
    )(t, sums, g2, beta2)


def kernel(x, edge_index, weight, W0, b0, g0, beta0, W1, b1, g1, beta1):
    n, f = x.shape
    nnz = edge_index.shape[1]
    eh = weight.shape[0]
    assert nnz % K == 0
    c = nnz // K

    src = edge_index[0].reshape(c, K)
    he = edge_index[1].reshape(c, K)

    dd = 16
    zeros_d = jnp.zeros((K, f), jnp.float32)
    zeros_16 = jnp.zeros((K, dd), jnp.float32)
    tab_w = jnp.zeros((eh, dd), jnp.float32).at[:, 0].set(weight)
    tab_1 = jnp.zeros((n, dd), jnp.float32).at[:, 0].set(1.0)

    seg_big_e = _seg_sum_rows(n, eh, c, f)
    seg_big_n = _seg_sum_rows(eh, n, c, f)
    seg_deg_n = _seg_sum_rows(eh, n, c, dd)
    seg_deg_e = _seg_sum_rows(n, eh, c, dd)

    dn_parts = seg_deg_n(tab_w, he, src, zeros_16)
    de_parts = seg_deg_e(tab_1, src, he, zeros_16)

    def layer(h, W, b, g, beta):
        xl = _matmul_t(h, W)
        pe = seg_big_e(xl, src, he, zeros_d)
        out_e = _combine_scale(pe, de_parts)
        pn = seg_big_n(out_e, he, src, zeros_d)
        t, sums = _combine_bias_tanh_stats(pn, dn_parts, b.reshape(1, f))
        return _batchnorm_apply(t, sums, g.reshape(1, f), beta.reshape(1, f))

    h1 = layer(x, W0, b0, g0, beta0)
    h2 = layer(h1, W1, b1, g1, beta1)
    return jnp.stack([h1, h2])


# speedup vs baseline: 4.3136x; 4.3136x over previous
import functools

import jax
import jax.numpy as jnp
from jax import lax
from jax.experimental import pallas as pl
from jax.experimental.pallas import tpu as pltpu
from jax.experimental.pallas import tpu_sc as plsc

EPS = 1e-5
K = 128


def _sc_dims():
    try:
        info = plsc.get_sparse_core_info()
        return info.num_cores, info.num_subcores
    except ValueError:
        return 2, 16


@functools.lru_cache(maxsize=None)
def _seg_sum_rows(T, S, CW, D, col_split):
    NC, NS = _sc_dims()
    assert S % 8 == 0
    DW = D // 2 if col_split else D
    base_rows = (S // NS) // 8 * 8
    tail = S - NS * base_rows
    mesh = plsc.VectorSubcoreMesh(core_axis_name="c", subcore_axis_name="s",
                                  num_cores=NC, num_subcores=NS)
    gshape = (NC, NS, CW, K) if col_split else (NC * NS, CW, K)
    sshape = (NS, CW, K) if col_split else (NC * NS, CW, K)

    @functools.partial(
        pl.kernel,
        out_type=jax.ShapeDtypeStruct((NC, S, DW), jnp.float32),
        mesh=mesh,
        scratch_types=[
            pltpu.VMEM((CW, K), jnp.int32),
            pltpu.VMEM((CW, K), jnp.int32),
            pltpu.VMEM((K, DW), jnp.float32),
            pltpu.VMEM((K, DW), jnp.float32),
            pltpu.VMEM_SHARED((S + K, DW), jnp.float32),
            pltpu.SemaphoreType.DMA,
        ],
        compiler_params=pltpu.CompilerParams(use_tc_tiling_on_sc=False),
    )
    def k(tab, gidx, sidx, zeros, out, gbuf, sbuf, rowbuf, zbuf, acc, sem):
        c = lax.axis_index("c")
        s = lax.axis_index("s")

        pltpu.sync_copy(zeros, zbuf)
        row0 = s * base_rows
        for off in range(0, base_rows, K):
            sz = min(K, base_rows - off)
            pltpu.sync_copy(zbuf.at[pl.ds(0, sz)], acc.at[pl.ds(row0 + off, sz)])
        if tail:
            @pl.when(s == NS - 1)
            def _():
                pltpu.sync_copy(zbuf.at[pl.ds(0, tail)],
                                acc.at[pl.ds(NS * base_rows, tail)])
        plsc.subcore_barrier()

        if col_split:
            pltpu.sync_copy(gidx.at[c, s], gbuf)
            pltpu.sync_copy(sidx.at[s], sbuf)
        else:
            w = s * NC + c
            pltpu.sync_copy(gidx.at[w], gbuf)
            pltpu.sync_copy(sidx.at[w], sbuf)

        def body(j, carry):
            pltpu.async_copy(tab.at[gbuf.at[j]], rowbuf, sem).wait()
            pltpu.sync_copy(rowbuf, acc.at[sbuf.at[j]], add=True)
            return carry

        lax.fori_loop(0, CW, body, 0)
        plsc.subcore_barrier()

        def wout(r0, sz):
            pltpu.sync_copy(acc.at[pl.ds(r0, sz)], rowbuf.at[pl.ds(0, sz)])
            pltpu.sync_copy(rowbuf.at[pl.ds(0, sz)], out.at[c, pl.ds(r0, sz)])

        for off in range(0, base_rows, K):
            wout(row0 + off, min(K, base_rows - off))
        if tail:
            @pl.when(s == NS - 1)
            def _():
                wout(NS * base_rows, tail)

    return k


def _safe_inv(d):
    return jnp.where(d > 0, 1.0 / jnp.where(d > 0, d, 1.0), 0.0)


def _matmul_t(x, W):
    n, f = x.shape
    blk = 1000
    assert n % blk == 0

    def body(x_ref, w_ref, o_ref):
        o_ref[...] = lax.dot_general(
            x_ref[...], w_ref[...], (((1,), (1,)), ((), ())),
            preferred_element_type=jnp.float32,
            precision=lax.Precision.HIGHEST)

    return pl.pallas_call(
        body,
        grid=(n // blk,),
        in_specs=[pl.BlockSpec((blk, f), lambda i: (i, 0)),
                  pl.BlockSpec((f, f), lambda i: (0, 0))],
        out_specs=pl.BlockSpec((blk, f), lambda i: (i, 0)),
        out_shape=jax.ShapeDtypeStruct((n, f), jnp.float32),
    )(x, W)


def _combine_scale(halves, deg_parts):
    _, s, dw = halves.shape
    d = 2 * dw
    dd = deg_parts.shape[2]
    blk = 1000
    assert s % blk == 0

    def body(p_ref, dg_ref, o_ref):
        deg = dg_ref[0, :, 0] + dg_ref[1, :, 0]
        full = jnp.concatenate([p_ref[0], p_ref[1]], axis=1)
        o_ref[...] = full * _safe_inv(deg)[:, None]

    return pl.pallas_call(
        body,
        grid=(s // blk,),
        in_specs=[pl.BlockSpec((2, blk, dw), lambda i: (0, i, 0)),
                  pl.BlockSpec((2, blk, dd), lambda i: (0, i, 0))],
        out_specs=pl.BlockSpec((blk, d), lambda i: (i, 0)),
        out_shape=jax.ShapeDtypeStruct((s, d), jnp.float32),
    )(halves, deg_parts)


def _combine_bias_tanh_stats(halves, deg_parts, b2):
    _, n, dw = halves.shape
    d = 2 * dw
    dd = deg_parts.shape[2]
    blk = 1000
    assert n % blk == 0

    def body(p_ref, dg_ref, b_ref, t_ref, s_ref):
        i = pl.program_id(0)
        deg = dg_ref[0, :, 0] + dg_ref[1, :, 0]
        full = jnp.concatenate([p_ref[0], p_ref[1]], axis=1)
        z = full * _safe_inv(deg)[:, None] + b_ref[...]
        t = jnp.tanh(z)
        t_ref[...] = t
        st = jnp.concatenate(
            [jnp.sum(t, 0, keepdims=True),
             jnp.sum(t * t, 0, keepdims=True),
             jnp.zeros((6, d), jnp.float32)], axis=0)

        @pl.when(i == 0)
        def _():
            s_ref[...] = jnp.zeros_like(s_ref)

        s_ref[...] = s_ref[...] + st

    return pl.pallas_call(
        body,
        grid=(n // blk,),
        in_specs=[pl.BlockSpec((2, blk, dw), lambda i: (0, i, 0)),
                  pl.BlockSpec((2, blk, dd), lambda i: (0, i, 0)),
                  pl.BlockSpec((1, d), lambda i: (0, 0))],
        out_specs=[pl.BlockSpec((blk, d), lambda i: (i, 0)),
                   pl.BlockSpec((8, d), lambda i: (0, 0))],
        out_shape=[jax.ShapeDtypeStruct((n, d), jnp.float32),
                   jax.ShapeDtypeStruct((8, d), jnp.float32)],
    )(halves, deg_parts, b2)


def _batchnorm_apply(t, sums, g2, beta2):
    n, d = t.shape
    blk = 1000
    assert n % blk == 0
    inv_n = 1.0 / n

    def body(t_ref, s_ref, g_ref, be_ref, o_ref):
        m = s_ref[0, :] * inv_n
        v = s_ref[1, :] * inv_n - m * m
        scale = lax.rsqrt(v + EPS) * g_ref[0, :]
        o_ref[...] = (t_ref[...] - m[None, :]) * scale[None, :] + be_ref[...]

    return pl.pallas_call(
        body,
        grid=(n // blk,),
        in_specs=[pl.BlockSpec((blk, d), lambda i: (i, 0)),
                  pl.BlockSpec((8, d), lambda i: (0, 0)),
                  pl.BlockSpec((1, d), lambda i: (0, 0)),
                  pl.BlockSpec((1, d), lambda i: (0, 0))],
        out_specs=pl.BlockSpec((blk, d), lambda i: (i, 0)),
        out_shape=jax.ShapeDtypeStruct((n, d), jnp.float32),
    )(t, sums, g2, beta2)


def kernel(x, edge_index, weight, W0, b0, g0, beta0, W1, b1, g1, beta1):
    n, f = x.shape
    nnz = edge_index.shape[1]
    eh = weight.shape[0]
    fw = f // 2
    NC, NS = _sc_dims()
    NW = NC * NS

    def pad_to(idx, nchunks, val):
        npad = nchunks * K - nnz
        return jnp.concatenate([idx, jnp.full((npad,), val, jnp.int32)])

    cw16 = -(-nnz // (NS * K))
    src16_g = pad_to(edge_index[0], NS * cw16, 0).reshape(NS, cw16, K)
    src16_s = pad_to(edge_index[0], NS * cw16, n).reshape(NS, cw16, K)
    he16_g = pad_to(edge_index[1], NS * cw16, 0).reshape(NS, cw16, K)
    he16_s = pad_to(edge_index[1], NS * cw16, eh).reshape(NS, cw16, K)
    src_cg = jnp.stack([2 * src16_g, 2 * src16_g + 1])
    he_cg = jnp.stack([2 * he16_g, 2 * he16_g + 1])

    zeros_h = jnp.zeros((K, fw), jnp.float32)
    tab_w = jnp.zeros((eh, f), jnp.float32).at[:, 0].set(weight)
    tab_1 = jnp.zeros((n, f), jnp.float32).at[:, 0].set(1.0)

    seg = _seg_sum_rows(n, eh, cw16, f, True)

    dn_parts = seg(tab_w.reshape(2 * eh, fw), he_cg, src16_s, zeros_h)
    de_parts = seg(tab_1.reshape(2 * n, fw), src_cg, he16_s, zeros_h)

    def layer(h, W, b, g, beta):
        xl = _matmul_t(h, W)
        pe = seg(xl.reshape(2 * n, fw), src_cg, he16_s, zeros_h)
        out_e = _combine_scale(pe, de_parts)
        pn = seg(out_e.reshape(2 * eh, fw), he_cg, src16_s, zeros_h)
        t, sums = _combine_bias_tanh_stats(pn, dn_parts, b.reshape(1, f))
        return _batchnorm_apply(t, sums, g.reshape(1, f), beta.reshape(1, f))

    h1 = layer(x, W0, b0, g0, beta0)
    h2 = layer(h1, W1, b1, g1, beta1)
    return jnp.stack([h1, h2])


# speedup vs baseline: 6.7859x; 1.5731x over previous
import functools

import jax
import jax.numpy as jnp
from jax import lax
from jax.experimental import pallas as pl
from jax.experimental.pallas import tpu as pltpu
from jax.experimental.pallas import tpu_sc as plsc

EPS = 1e-5
K = 128


def _sc_dims():
    try:
        info = plsc.get_sparse_core_info()
        return info.num_cores, info.num_subcores
    except ValueError:
        return 2, 16


@functools.lru_cache(maxsize=None)
def _seg_sum_rows(T, S, CW, D, col_split):
    NC, NS = _sc_dims()
    assert S % 8 == 0
    DW = D // 2 if col_split else D
    base_rows = (S // NS) // 8 * 8
    tail = S - NS * base_rows
    mesh = plsc.VectorSubcoreMesh(core_axis_name="c", subcore_axis_name="s",
                                  num_cores=NC, num_subcores=NS)
    gshape = (NC, NS, CW, K) if col_split else (NC * NS, CW, K)
    sshape = (NS, CW, K) if col_split else (NC * NS, CW, K)

    @functools.partial(
        pl.kernel,
        out_type=jax.ShapeDtypeStruct((NC, S, DW), jnp.float32),
        mesh=mesh,
        scratch_types=[
            pltpu.VMEM((CW, K), jnp.int32),
            pltpu.VMEM((CW, K), jnp.int32),
            pltpu.VMEM((K, DW), jnp.float32),
            pltpu.VMEM((K, DW), jnp.float32),
            pltpu.VMEM((K, DW), jnp.float32),
            pltpu.VMEM_SHARED((S + K, DW), jnp.float32),
            pltpu.SemaphoreType.DMA,
        ],
        compiler_params=pltpu.CompilerParams(use_tc_tiling_on_sc=False),
    )
    def k(tab, gidx, sidx, zeros, out, gbuf, sbuf, rowbuf, rowbuf2, zbuf, acc, sem):
        c = lax.axis_index("c")
        s = lax.axis_index("s")

        pltpu.sync_copy(zeros, zbuf)
        row0 = s * base_rows
        for off in range(0, base_rows, K):
            sz = min(K, base_rows - off)
            pltpu.sync_copy(zbuf.at[pl.ds(0, sz)], acc.at[pl.ds(row0 + off, sz)])
        if tail:
            @pl.when(s == NS - 1)
            def _():
                pltpu.sync_copy(zbuf.at[pl.ds(0, tail)],
                                acc.at[pl.ds(NS * base_rows, tail)])
        plsc.subcore_barrier()

        if col_split:
            pltpu.sync_copy(gidx.at[c, s], gbuf)
            pltpu.sync_copy(sidx.at[s], sbuf)
        else:
            w = s * NC + c
            pltpu.sync_copy(gidx.at[w], gbuf)
            pltpu.sync_copy(sidx.at[w], sbuf)

        def start_g(j, buf):
            pltpu.async_copy(tab.at[gbuf.at[j]], buf, sem)

        def wait_g(j, buf):
            pltpu.make_async_copy(tab.at[gbuf.at[j]], buf, sem).wait()

        def scat(j, buf):
            pltpu.sync_copy(buf, acc.at[sbuf.at[j]], add=True)

        start_g(0, rowbuf)

        def body(i, carry):
            j = 2 * i
            wait_g(j, rowbuf)
            start_g(j + 1, rowbuf2)
            scat(j, rowbuf)
            wait_g(j + 1, rowbuf2)
            start_g(j + 2, rowbuf)
            scat(j + 1, rowbuf2)
            return carry

        lax.fori_loop(0, (CW - 1) // 2, body, 0)
        wait_g(CW - 1, rowbuf)
        scat(CW - 1, rowbuf)
        plsc.subcore_barrier()

        def wout(r0, sz):
            pltpu.sync_copy(acc.at[pl.ds(r0, sz)], rowbuf.at[pl.ds(0, sz)])
            pltpu.sync_copy(rowbuf.at[pl.ds(0, sz)], out.at[c, pl.ds(r0, sz)])

        for off in range(0, base_rows, K):
            wout(row0 + off, min(K, base_rows - off))
        if tail:
            @pl.when(s == NS - 1)
            def _():
                wout(NS * base_rows, tail)

    return k


@functools.lru_cache(maxsize=None)
def _degrees(NN, EE, CW):
    NC, NS = _sc_dims()
    NW = NC * NS
    DD = 16
    assert CW % 2 == 1

    def plan(S):
        base_rows = (S // NS) // 8 * 8
        return base_rows, S - NS * base_rows

    mesh = plsc.VectorSubcoreMesh(core_axis_name="c", subcore_axis_name="s",
                                  num_cores=NC, num_subcores=NS)

    @functools.partial(
        pl.kernel,
        out_type=(jax.ShapeDtypeStruct((NC, NN, DD), jnp.float32),
                  jax.ShapeDtypeStruct((NC, EE, DD), jnp.float32)),
        mesh=mesh,
        scratch_types=[
            pltpu.VMEM((CW, K), jnp.int32),
            pltpu.VMEM((CW, K), jnp.int32),
            pltpu.VMEM((K, DD), jnp.float32),
            pltpu.VMEM((K, DD), jnp.float32),
            pltpu.VMEM((K, DD), jnp.float32),
            pltpu.VMEM((K, DD), jnp.float32),
            pltpu.VMEM_SHARED((NN + K, DD), jnp.float32),
            pltpu.VMEM_SHARED((EE + K, DD), jnp.float32),
            pltpu.SemaphoreType.DMA,
        ],
        compiler_params=pltpu.CompilerParams(use_tc_tiling_on_sc=False),
    )
    def k(wtab, hidx, sidx, ones, zeros, dn, de, hbuf, sbuf, rb0, rb1, onesb,
          zbuf, accn, acce, sem):
        c = lax.axis_index("c")
        s = lax.axis_index("s")
        w = s * NC + c

        pltpu.sync_copy(zeros, zbuf)
        pltpu.sync_copy(ones, onesb)
        for acc, S in ((accn, NN), (acce, EE)):
            base_rows, tail = plan(S)
            row0 = s * base_rows
            for off in range(0, base_rows, K):
                sz = min(K, base_rows - off)
                pltpu.sync_copy(zbuf.at[pl.ds(0, sz)], acc.at[pl.ds(row0 + off, sz)])
            if tail:
                @pl.when(s == NS - 1)
                def _():
                    pltpu.sync_copy(zbuf.at[pl.ds(0, tail)],
                                    acc.at[pl.ds(NS * base_rows, tail)])
        plsc.subcore_barrier()

        pltpu.sync_copy(hidx.at[w], hbuf)
        pltpu.sync_copy(sidx.at[w], sbuf)

        def start_g(j, buf):
            pltpu.async_copy(wtab.at[hbuf.at[j]], buf, sem)

        def wait_g(j, buf):
            pltpu.make_async_copy(wtab.at[hbuf.at[j]], buf, sem).wait()

        def scat(j, buf):
            pltpu.sync_copy(buf, accn.at[sbuf.at[j]], add=True)
            pltpu.sync_copy(onesb, acce.at[hbuf.at[j]], add=True)

        start_g(0, rb0)

        def body(i, carry):
            j = 2 * i
            wait_g(j, rb0)
            start_g(j + 1, rb1)
            scat(j, rb0)
            wait_g(j + 1, rb1)
            start_g(j + 2, rb0)
            scat(j + 1, rb1)
            return carry

        lax.fori_loop(0, (CW - 1) // 2, body, 0)
        wait_g(CW - 1, rb0)
        scat(CW - 1, rb0)
        plsc.subcore_barrier()

        for acc, S, out in ((accn, NN, dn), (acce, EE, de)):
            base_rows, tail = plan(S)
            row0 = s * base_rows

            def wout(r0, sz, acc=acc, out=out):
                pltpu.sync_copy(acc.at[pl.ds(r0, sz)], rb0.at[pl.ds(0, sz)])
                pltpu.sync_copy(rb0.at[pl.ds(0, sz)], out.at[c, pl.ds(r0, sz)])

            for off in range(0, base_rows, K):
                wout(row0 + off, min(K, base_rows - off))
            if tail:
                @pl.when(s == NS - 1)
                def _():
                    wout(NS * base_rows, tail)

    return k


def _safe_inv(d):
    return jnp.where(d > 0, 1.0 / jnp.where(d > 0, d, 1.0), 0.0)


def _matmul_t(x, W):
    n, f = x.shape
    blk = 1000
    assert n % blk == 0

    def body(x_ref, w_ref, o_ref):
        o_ref[...] = lax.dot_general(
            x_ref[...], w_ref[...], (((1,), (1,)), ((), ())),
            preferred_element_type=jnp.float32,
            precision=lax.Precision.HIGHEST)

    return pl.pallas_call(
        body,
        grid=(n // blk,),
        in_specs=[pl.BlockSpec((blk, f), lambda i: (i, 0)),
                  pl.BlockSpec((f, f), lambda i: (0, 0))],
        out_specs=pl.BlockSpec((blk, f), lambda i: (i, 0)),
        out_shape=jax.ShapeDtypeStruct((n, f), jnp.float32),
    )(x, W)


def _combine_scale(halves, deg_parts):
    _, s, dw = halves.shape
    d = 2 * dw
    dd = deg_parts.shape[2]
    blk = 1000
    assert s % blk == 0

    def body(p_ref, dg_ref, o_ref):
        deg = dg_ref[0, :, 0] + dg_ref[1, :, 0]
        full = jnp.concatenate([p_ref[0], p_ref[1]], axis=1)
        o_ref[...] = full * _safe_inv(deg)[:, None]

    return pl.pallas_call(
        body,
        grid=(s // blk,),
        in_specs=[pl.BlockSpec((2, blk, dw), lambda i: (0, i, 0)),
                  pl.BlockSpec((2, blk, dd), lambda i: (0, i, 0))],
        out_specs=pl.BlockSpec((blk, d), lambda i: (i, 0)),
        out_shape=jax.ShapeDtypeStruct((s, d), jnp.float32),
    )(halves, deg_parts)


def _combine_bias_tanh_stats(halves, deg_parts, b2):
    _, n, dw = halves.shape
    d = 2 * dw
    dd = deg_parts.shape[2]
    blk = 1000
    assert n % blk == 0

    def body(p_ref, dg_ref, b_ref, t_ref, s_ref):
        i = pl.program_id(0)
        deg = dg_ref[0, :, 0] + dg_ref[1, :, 0]
        full = jnp.concatenate([p_ref[0], p_ref[1]], axis=1)
        z = full * _safe_inv(deg)[:, None] + b_ref[...]
        t = jnp.tanh(z)
        t_ref[...] = t
        st = jnp.concatenate(
            [jnp.sum(t, 0, keepdims=True),
             jnp.sum(t * t, 0, keepdims=True),
             jnp.zeros((6, d), jnp.float32)], axis=0)

        @pl.when(i == 0)
        def _():
            s_ref[...] = jnp.zeros_like(s_ref)

        s_ref[...] = s_ref[...] + st

    return pl.pallas_call(
        body,
        grid=(n // blk,),
        in_specs=[pl.BlockSpec((2, blk, dw), lambda i: (0, i, 0)),
                  pl.BlockSpec((2, blk, dd), lambda i: (0, i, 0)),
                  pl.BlockSpec((1, d), lambda i: (0, 0))],
        out_specs=[pl.BlockSpec((blk, d), lambda i: (i, 0)),
                   pl.BlockSpec((8, d), lambda i: (0, 0))],
        out_shape=[jax.ShapeDtypeStruct((n, d), jnp.float32),
                   jax.ShapeDtypeStruct((8, d), jnp.float32)],
    )(halves, deg_parts, b2)


def _batchnorm_apply(t, sums, g2, beta2):
    n, d = t.shape
    blk = 1000
    assert n % blk == 0
    inv_n = 1.0 / n

    def body(t_ref, s_ref, g_ref, be_ref, o_ref):
        m = s_ref[0, :] * inv_n
        v = s_ref[1, :] * inv_n - m * m
        scale = lax.rsqrt(v + EPS) * g_ref[0, :]
        o_ref[...] = (t_ref[...] - m[None, :]) * scale[None, :] + be_ref[...]

    return pl.pallas_call(
        body,
        grid=(n // blk,),
        in_specs=[pl.BlockSpec((blk, d), lambda i: (i, 0)),
                  pl.BlockSpec((8, d), lambda i: (0, 0)),
                  pl.BlockSpec((1, d), lambda i: (0, 0)),
                  pl.BlockSpec((1, d), lambda i: (0, 0))],
        out_specs=pl.BlockSpec((blk, d), lambda i: (i, 0)),
        out_shape=jax.ShapeDtypeStruct((n, d), jnp.float32),
    )(t, sums, g2, beta2)


def kernel(x, edge_index, weight, W0, b0, g0, beta0, W1, b1, g1, beta1):
    n, f = x.shape
    nnz = edge_index.shape[1]
    eh = weight.shape[0]
    fw = f // 2
    NC, NS = _sc_dims()
    NW = NC * NS

    def pad_to(idx, nchunks, val):
        npad = nchunks * K - nnz
        return jnp.concatenate([idx, jnp.full((npad,), val, jnp.int32)])

    cw16 = -(-nnz // (NS * K))
    cw16 += 1 - cw16 % 2
    src16_g = pad_to(edge_index[0], NS * cw16, 0).reshape(NS, cw16, K)
    src16_s = pad_to(edge_index[0], NS * cw16, n).reshape(NS, cw16, K)
    he16_g = pad_to(edge_index[1], NS * cw16, 0).reshape(NS, cw16, K)
    he16_s = pad_to(edge_index[1], NS * cw16, eh).reshape(NS, cw16, K)
    src_cg = jnp.stack([2 * src16_g, 2 * src16_g + 1])
    he_cg = jnp.stack([2 * he16_g, 2 * he16_g + 1])

    zeros_h = jnp.zeros((K, fw), jnp.float32)

    cw32 = -(-nnz // (NW * K))
    cw32 += 1 - cw32 % 2
    he32 = pad_to(edge_index[1], NW * cw32, eh).reshape(NW, cw32, K)
    src32 = pad_to(edge_index[0], NW * cw32, n).reshape(NW, cw32, K)
    tab_w16 = jnp.zeros((eh + K, 16), jnp.float32).at[:eh, 0].set(weight)
    ones16 = jnp.ones((K, 16), jnp.float32)
    zeros16 = jnp.zeros((K, 16), jnp.float32)

    seg = _seg_sum_rows(n, eh, cw16, f, True)

    dn_parts, de_parts = _degrees(n, eh, cw32)(
        tab_w16, he32, src32, ones16, zeros16)

    def layer(h, W, b, g, beta):
        xl = _matmul_t(h, W)
        pe = seg(xl.reshape(2 * n, fw), src_cg, he16_s, zeros_h)
        out_e = _combine_scale(pe, de_parts)
        pn = seg(out_e.reshape(2 * eh, fw), he_cg, src16_s, zeros_h)
        t, sums = _combine_bias_tanh_stats(pn, dn_parts, b.reshape(1, f))
        return _batchnorm_apply(t, sums, g.reshape(1, f), beta.reshape(1, f))

    h1 = layer(x, W0, b0, g0, beta0)
    h2 = layer(h1, W1, b1, g1, beta1)
    return jnp.stack([h1, h2])
